# baseline (speedup 1.0000x reference)
import functools

import jax
import jax.numpy as jnp
from jax import lax
from jax.experimental import pallas as pl
from jax.experimental.pallas import tpu as pltpu
from jax.experimental.pallas import tpu_sc as plsc

N_NODES = 10000
D = 128
DH = D // 2
N_EDGES = 320000
NC = 2
NS = 16
NW = NC * NS
EBH = 80
NBH = N_EDGES // NW // EBH
EBA = 125
NBA = N_EDGES // NS // EBA
RPT = N_NODES // NS
RC = 125
NCH = RPT // RC
NHIST = 10240
HPT = NHIST // NS

_MESH = plsc.VectorSubcoreMesh(core_axis_name="c", subcore_axis_name="s")


@functools.partial(
    pl.kernel,
    out_type=jax.ShapeDtypeStruct((NC * NHIST,), jnp.float32),
    mesh=_MESH,
    scratch_types=[
        pltpu.VMEM((NBH, EBH), jnp.int32),
        pltpu.VMEM((EBH,), jnp.float32),
        pltpu.VMEM((HPT,), jnp.float32),
        pltpu.VMEM_SHARED((NHIST,), jnp.float32),
    ],
)
def _sc_degree(dst_hbm, out_hbm, didx_v, ones_v, buf_v, acc_sh):
    cid = lax.axis_index("c")
    sid = lax.axis_index("s")
    wid = cid * NS + sid
    for k in range(EBH // 16):
        ones_v[pl.ds(k * 16, 16)] = jnp.ones((16,), jnp.float32)
    for k in range(HPT // 16):
        buf_v[pl.ds(k * 16, 16)] = jnp.zeros((16,), jnp.float32)
    pltpu.sync_copy(buf_v, acc_sh.at[pl.ds(sid * HPT, HPT)])
    plsc.subcore_barrier()
    pltpu.sync_copy(dst_hbm.at[wid], didx_v)

    def body(j, carry):
        pltpu.sync_copy(ones_v, acc_sh.at[didx_v.at[j]], add=True)
        return carry

    lax.fori_loop(0, NBH, body, 0)
    plsc.subcore_barrier()
    pltpu.sync_copy(acc_sh.at[pl.ds(sid * HPT, HPT)], buf_v)
    pltpu.sync_copy(buf_v, out_hbm.at[pl.ds(cid * NHIST + sid * HPT, HPT)])


NBUF = 10
LOOK = 8


@functools.partial(
    pl.kernel,
    out_type=jax.ShapeDtypeStruct((NW * NCH, RC, DH), jnp.bfloat16),
    mesh=_MESH,
    scratch_types=[
        pltpu.VMEM((NBA, EBA), jnp.int32),
        pltpu.VMEM((NBA, EBA), jnp.int32),
        pltpu.VMEM((NBUF, EBA, DH), jnp.bfloat16),
        pltpu.VMEM_SHARED((N_NODES, DH), jnp.bfloat16),
        pltpu.SemaphoreType.DMA,
        pltpu.SemaphoreType.DMA,
    ],
    compiler_params=pltpu.CompilerParams(use_tc_tiling_on_sc=False),
)
def _sc_aggregate(g_hbm, src_hbm, dst_hbm, z_hbm, out_hbm,
                  sidx_v, didx_v, rows_v, acc_sh, sem_g, sem_s):
    cid = lax.axis_index("c")
    sid = lax.axis_index("s")
    pltpu.sync_copy(z_hbm, acc_sh.at[pl.ds(sid * RPT, RPT)])
    plsc.subcore_barrier()
    pltpu.sync_copy(src_hbm.at[sid], sidx_v)
    pltpu.sync_copy(dst_hbm.at[sid], didx_v)

    def fire_gather(j, bank):
        pltpu.async_copy(g_hbm.at[cid].at[sidx_v.at[j]], rows_v.at[bank],
                         sem_g)

    def wait_gather(bank):
        pltpu.make_async_copy(g_hbm.at[cid].at[sidx_v.at[0]],
                              rows_v.at[bank], sem_g).wait()

    def fire_scatter(j, bank):
        pltpu.async_copy(rows_v.at[bank], acc_sh.at[didx_v.at[j]], sem_s,
                         add=True)

    def wait_scatter():
        pltpu.make_async_copy(rows_v.at[0], acc_sh.at[didx_v.at[0]],
                              sem_s).wait()

    for p in range(LOOK):
        fire_gather(p, p)

    def step(j, p, wait_s, fire_g):
        wait_gather(p % NBUF)
        fire_scatter(j, p % NBUF)
        if wait_s:
            wait_scatter()
        if fire_g:
            fire_gather(j + LOOK, (p + LOOK) % NBUF)

    for p in range(NBUF):
        step(p, p, wait_s=(p >= 2), fire_g=True)

    def steady(blk, carry):
        j0 = blk * NBUF
        for p in range(NBUF):
            step(j0 + p, p, wait_s=True, fire_g=True)
        return carry

    lax.fori_loop(1, NBA // NBUF - 1, steady, 0)

    for p in range(NBUF):
        j = NBA - NBUF + p
        step(j, p, wait_s=(j + LOOK < NBA + 2), fire_g=(j + LOOK < NBA))
    for _ in range(NBUF - 2):
        wait_scatter()

    plsc.subcore_barrier()
    for k in range(NCH):
        pltpu.sync_copy(acc_sh.at[pl.ds(sid * RPT + k * RC, RC)],
                        out_hbm.at[(cid * NS + sid) * NCH + k])


def _tc_first(d0, d1, x, W1):
    def body(d0_ref, d1_ref, x_ref, w_ref, dinv_ref, g_ref, gf_ref):
        dinv = lax.rsqrt(d0_ref[...] + d1_ref[...] + 1.0)
        dinv_ref[...] = dinv
        g = jnp.dot(x_ref[...] * dinv, w_ref[...],
                    preferred_element_type=jnp.float32)
        gb = g.astype(jnp.bfloat16)
        g_ref[0] = gb[:, :DH]
        g_ref[1] = gb[:, DH:]
        gf_ref[...] = g

    return pl.pallas_call(
        body,
        out_shape=(jax.ShapeDtypeStruct((N_NODES, 1), jnp.float32),
                   jax.ShapeDtypeStruct((NC, N_NODES, DH), jnp.bfloat16),
                   jax.ShapeDtypeStruct((N_NODES, D), jnp.float32)),
    )(d0, d1, x, W1)


def _tc_mid(agg, gf, dinv, b1, W2):
    def body(a_ref, gf_ref, dinv_ref, b_ref, w_ref, g2_ref, g2f_ref):
        dinv = dinv_ref[...]
        agg_f = jnp.concatenate(
            [a_ref[0].astype(jnp.float32), a_ref[1].astype(jnp.float32)],
            axis=1)
        h = jnp.maximum(dinv * (agg_f + gf_ref[...]) + b_ref[...], 0.0)
        g2 = jnp.dot(h * dinv, w_ref[...], preferred_element_type=jnp.float32)
        g2b = g2.astype(jnp.bfloat16)
        g2_ref[0] = g2b[:, :DH]
        g2_ref[1] = g2b[:, DH:]
        g2f_ref[...] = g2

    return pl.pallas_call(
        body,
        out_shape=(jax.ShapeDtypeStruct((NC, N_NODES, DH), jnp.bfloat16),
                   jax.ShapeDtypeStruct((N_NODES, D), jnp.float32)),
    )(agg, gf, dinv, b1, W2)


def _tc_last(agg, gf, dinv, b2):
    def body(a_ref, gf_ref, dinv_ref, b_ref, out_ref):
        agg_f = jnp.concatenate(
            [a_ref[0].astype(jnp.float32), a_ref[1].astype(jnp.float32)],
            axis=1)
        out_ref[...] = jnp.maximum(
            dinv_ref[...] * (agg_f + gf_ref[...]) + b_ref[...], 0.0)

    return pl.pallas_call(
        body,
        out_shape=jax.ShapeDtypeStruct((N_NODES, D), jnp.float32),
    )(agg, gf, dinv, b2)


def kernel(x, edge_index, W1, b1, W2, b2):
    src = edge_index[0].astype(jnp.int32).reshape(NS, NBA, EBA)
    dst = edge_index[1].astype(jnp.int32).reshape(NS, NBA, EBA)
    dsth = dst.reshape(NW, NBH, EBH)
    zeros = jnp.zeros((RPT, DH), jnp.bfloat16)

    degf = _sc_degree(dsth)
    d0 = degf[:N_NODES, None]
    d1 = degf[NHIST:NHIST + N_NODES, None]

    dinv, g1, g1f = _tc_first(d0, d1, x, W1)

    agg1 = _sc_aggregate(g1, src, dst, zeros).reshape(NC, N_NODES, DH)
    g2, g2f = _tc_mid(agg1, g1f, dinv, b1.reshape(1, D), W2)

    agg2 = _sc_aggregate(g2, src, dst, zeros).reshape(NC, N_NODES, DH)
    return _tc_last(agg2, g2f, dinv, b2.reshape(1, D))


# speedup vs baseline: 1.0156x; 1.0156x over previous
import functools

import jax
import jax.numpy as jnp
from jax import lax
from jax.experimental import pallas as pl
from jax.experimental.pallas import tpu as pltpu
from jax.experimental.pallas import tpu_sc as plsc

N_NODES = 10000
D = 128
DH = D // 2
N_EDGES = 320000
NC = 2
NS = 16
NW = NC * NS
EBH = 80
NBH = N_EDGES // NW // EBH
EBA = 125
NBA = N_EDGES // NS // EBA
RPT = N_NODES // NS
RC = 125
NCH = RPT // RC
NHIST = 10240
HPT = NHIST // NS

_MESH = plsc.VectorSubcoreMesh(core_axis_name="c", subcore_axis_name="s")


@functools.partial(
    pl.kernel,
    out_type=jax.ShapeDtypeStruct((NC * NHIST,), jnp.float32),
    mesh=_MESH,
    scratch_types=[
        pltpu.VMEM((NBH, EBH), jnp.int32),
        pltpu.VMEM((EBH,), jnp.float32),
        pltpu.VMEM((HPT,), jnp.float32),
        pltpu.VMEM_SHARED((NHIST,), jnp.float32),
    ],
)
def _sc_degree(dst_hbm, out_hbm, didx_v, ones_v, buf_v, acc_sh):
    cid = lax.axis_index("c")
    sid = lax.axis_index("s")
    wid = cid * NS + sid
    for k in range(EBH // 16):
        ones_v[pl.ds(k * 16, 16)] = jnp.ones((16,), jnp.float32)
    for k in range(HPT // 16):
        buf_v[pl.ds(k * 16, 16)] = jnp.zeros((16,), jnp.float32)
    pltpu.sync_copy(buf_v, acc_sh.at[pl.ds(sid * HPT, HPT)])
    plsc.subcore_barrier()
    pltpu.sync_copy(dst_hbm.at[wid], didx_v)

    def body(j, carry):
        pltpu.sync_copy(ones_v, acc_sh.at[didx_v.at[j]], add=True)
        return carry

    lax.fori_loop(0, NBH, body, 0)
    plsc.subcore_barrier()
    pltpu.sync_copy(acc_sh.at[pl.ds(sid * HPT, HPT)], buf_v)
    pltpu.sync_copy(buf_v, out_hbm.at[pl.ds(cid * NHIST + sid * HPT, HPT)])


NBUF = 8
LOOK = 6


@functools.partial(
    pl.kernel,
    out_type=jax.ShapeDtypeStruct((NW * NCH, RC, DH), jnp.bfloat16),
    mesh=_MESH,
    scratch_types=[
        pltpu.VMEM((NBA, EBA), jnp.int32),
        pltpu.VMEM((NBA, EBA), jnp.int32),
        pltpu.VMEM((NBUF, EBA, DH), jnp.bfloat16),
        pltpu.VMEM_SHARED((N_NODES, DH), jnp.bfloat16),
        pltpu.SemaphoreType.DMA,
        pltpu.SemaphoreType.DMA,
    ],
    compiler_params=pltpu.CompilerParams(use_tc_tiling_on_sc=False),
)
def _sc_aggregate(g_hbm, src_hbm, dst_hbm, z_hbm, out_hbm,
                  sidx_v, didx_v, rows_v, acc_sh, sem_g, sem_s):
    cid = lax.axis_index("c")
    sid = lax.axis_index("s")
    pltpu.sync_copy(z_hbm, acc_sh.at[pl.ds(sid * RPT, RPT)])
    plsc.subcore_barrier()
    pltpu.sync_copy(src_hbm.at[sid], sidx_v)
    pltpu.sync_copy(dst_hbm.at[sid], didx_v)

    def fire_gather(j, bank):
        pltpu.async_copy(g_hbm.at[cid].at[sidx_v.at[j]], rows_v.at[bank],
                         sem_g)

    def wait_gather(bank):
        pltpu.make_async_copy(g_hbm.at[cid].at[sidx_v.at[0]],
                              rows_v.at[bank], sem_g).wait()

    def fire_scatter(j, bank):
        pltpu.async_copy(rows_v.at[bank], acc_sh.at[didx_v.at[j]], sem_s,
                         add=True)

    def wait_scatter():
        pltpu.make_async_copy(rows_v.at[0], acc_sh.at[didx_v.at[0]],
                              sem_s).wait()

    for p in range(LOOK):
        fire_gather(p, p)

    def step(j, p, wait_s, fire_g):
        wait_gather(p % NBUF)
        fire_scatter(j, p % NBUF)
        if wait_s:
            wait_scatter()
        if fire_g:
            fire_gather(j + LOOK, (p + LOOK) % NBUF)

    for p in range(NBUF):
        step(p, p, wait_s=(p >= 2), fire_g=True)

    def steady(blk, carry):
        j0 = blk * NBUF
        for p in range(NBUF):
            step(j0 + p, p, wait_s=True, fire_g=True)
        return carry

    lax.fori_loop(1, NBA // NBUF - 1, steady, 0)

    for p in range(NBUF):
        j = NBA - NBUF + p
        step(j, p, wait_s=(j + LOOK < NBA + 2), fire_g=(j + LOOK < NBA))
    for _ in range(NBUF - 2):
        wait_scatter()

    plsc.subcore_barrier()
    for k in range(NCH):
        pltpu.sync_copy(acc_sh.at[pl.ds(sid * RPT + k * RC, RC)],
                        out_hbm.at[(cid * NS + sid) * NCH + k])


def _tc_first(degf, x, W1):
    def body(deg_ref, x_ref, w_ref, dinv_ref, g_ref, gf_ref):
        d0 = deg_ref[pl.ds(0, N_NODES)]
        d1 = deg_ref[pl.ds(NHIST, N_NODES)]
        dinv = lax.rsqrt(d0 + d1 + 1.0)
        dinv_ref[...] = dinv
        g = jnp.dot(x_ref[...] * dinv, w_ref[...],
                    preferred_element_type=jnp.float32)
        gb = g.astype(jnp.bfloat16)
        g_ref[0] = gb[:, :DH]
        g_ref[1] = gb[:, DH:]
        gf_ref[...] = g

    return pl.pallas_call(
        body,
        out_shape=(jax.ShapeDtypeStruct((N_NODES, 1), jnp.float32),
                   jax.ShapeDtypeStruct((NC, N_NODES, DH), jnp.bfloat16),
                   jax.ShapeDtypeStruct((N_NODES, D), jnp.float32)),
    )(degf, x, W1)


def _tc_mid(agg, gf, dinv, b1, W2):
    def body(a_ref, gf_ref, dinv_ref, b_ref, w_ref, g2_ref, g2f_ref):
        dinv = dinv_ref[...]
        agg_f = jnp.concatenate(
            [a_ref[0].astype(jnp.float32), a_ref[1].astype(jnp.float32)],
            axis=1)
        h = jnp.maximum(dinv * (agg_f + gf_ref[...]) + b_ref[...], 0.0)
        g2 = jnp.dot(h * dinv, w_ref[...], preferred_element_type=jnp.float32)
        g2b = g2.astype(jnp.bfloat16)
        g2_ref[0] = g2b[:, :DH]
        g2_ref[1] = g2b[:, DH:]
        g2f_ref[...] = g2

    return pl.pallas_call(
        body,
        out_shape=(jax.ShapeDtypeStruct((NC, N_NODES, DH), jnp.bfloat16),
                   jax.ShapeDtypeStruct((N_NODES, D), jnp.float32)),
    )(agg, gf, dinv, b1, W2)


def _tc_last(agg, gf, dinv, b2):
    def body(a_ref, gf_ref, dinv_ref, b_ref, out_ref):
        agg_f = jnp.concatenate(
            [a_ref[0].astype(jnp.float32), a_ref[1].astype(jnp.float32)],
            axis=1)
        out_ref[...] = jnp.maximum(
            dinv_ref[...] * (agg_f + gf_ref[...]) + b_ref[...], 0.0)

    return pl.pallas_call(
        body,
        out_shape=jax.ShapeDtypeStruct((N_NODES, D), jnp.float32),
    )(agg, gf, dinv, b2)


def kernel(x, edge_index, W1, b1, W2, b2):
    src = edge_index[0].astype(jnp.int32).reshape(NS, NBA, EBA)
    dst = edge_index[1].astype(jnp.int32).reshape(NS, NBA, EBA)
    dsth = dst.reshape(NW, NBH, EBH)
    zeros = jnp.zeros((RPT, DH), jnp.bfloat16)

    degf = _sc_degree(dsth)

    dinv, g1, g1f = _tc_first(degf[:, None], x, W1)

    agg1 = _sc_aggregate(g1, src, dst, zeros).reshape(NC, N_NODES, DH)
    g2, g2f = _tc_mid(agg1, g1f, dinv, b1.reshape(1, D), W2)

    agg2 = _sc_aggregate(g2, src, dst, zeros).reshape(NC, N_NODES, DH)
    return _tc_last(agg2, g2f, dinv, b2.reshape(1, D))
